# baseline (speedup 1.0000x reference)
import functools

import jax
import jax.numpy as jnp
from jax import lax
from jax.experimental import pallas as pl
from jax.experimental.pallas import tpu as pltpu
from jax.experimental.pallas import tpu_sc as plsc

B = 4
T = 4096
D = 1024
NC = 2
NS = 16
L = 16
NW = NC * NS
ROWS_PER_W = (B * T) // NW
CH = 64
NCH = ROWS_PER_W // CH

_mesh = plsc.VectorSubcoreMesh(
    core_axis_name="c", subcore_axis_name="s", num_cores=NC, num_subcores=NS
)


@functools.partial(
    pl.kernel,
    out_type=jax.ShapeDtypeStruct((B * T, D), jnp.float32),
    mesh=_mesh,
    compiler_params=pltpu.CompilerParams(
        use_tc_tiling_on_sc=False, needs_layout_passes=False
    ),
    scratch_types=[
        pltpu.VMEM((L,), jnp.int32),
        pltpu.VMEM((NCH, CH), jnp.int32),
        pltpu.VMEM((CH, D), jnp.float32),
        pltpu.SemaphoreType.DMA,
    ],
)
def _sc_gather(off_hbm, weight_hbm, out_hbm, off_v, idx_v, buf, sem):
    wid = lax.axis_index("s") * NC + lax.axis_index("c")
    b = wid // (NW // B)
    t0 = (wid % (NW // B)) * ROWS_PER_W

    pltpu.sync_copy(off_hbm, off_v)
    off_vec = off_v[...]
    sel = jnp.where(lax.iota(jnp.int32, L) == b, off_vec, 0)
    base = jnp.sum(sel) + t0

    pltpu.async_copy(
        weight_hbm.at[pl.ds(base, ROWS_PER_W)],
        out_hbm.at[pl.ds(wid * ROWS_PER_W, ROWS_PER_W)],
        sem,
    ).wait()


def kernel(length, offset, weight):
    del length
    off = jnp.ravel(offset).astype(jnp.int32)
    off_pad = jnp.zeros((L,), jnp.int32).at[:B].set(off)
    out = _sc_gather(off_pad, weight)
    return out.reshape(B, T, D)


# speedup vs baseline: 13.1465x; 13.1465x over previous
import functools

import jax
import jax.numpy as jnp
from jax import lax
from jax.experimental import pallas as pl
from jax.experimental.pallas import tpu as pltpu
from jax.experimental.pallas import tpu_sc as plsc

B = 4
T = 4096
D = 1024
NC = 2
NS = 16
L = 16
NW = NC * NS
ROWS_PER_W = (B * T) // NW
CH = 32
NCH = ROWS_PER_W // CH

_mesh = plsc.VectorSubcoreMesh(
    core_axis_name="c", subcore_axis_name="s", num_cores=NC, num_subcores=NS
)


@functools.partial(
    pl.kernel,
    out_type=jax.ShapeDtypeStruct((B * T, D), jnp.float32),
    mesh=_mesh,
    compiler_params=pltpu.CompilerParams(
        use_tc_tiling_on_sc=False, needs_layout_passes=False
    ),
    scratch_types=[
        pltpu.VMEM((L,), jnp.int32),
        pltpu.VMEM((CH, D), jnp.float32),
        pltpu.VMEM((CH, D), jnp.float32),
        pltpu.SemaphoreType.DMA,
        pltpu.SemaphoreType.DMA,
        pltpu.SemaphoreType.DMA,
        pltpu.SemaphoreType.DMA,
    ],
)
def _sc_gather(off_hbm, weight_hbm, out_hbm, off_v, buf0, buf1, sg0, sg1, ss0, ss1):
    wid = lax.axis_index("s") * NC + lax.axis_index("c")
    b = wid // (NW // B)
    t0 = (wid % (NW // B)) * ROWS_PER_W

    pltpu.sync_copy(off_hbm, off_v)
    off_vec = off_v[...]
    sel = jnp.where(lax.iota(jnp.int32, L) == b, off_vec, 0)
    base = jnp.sum(sel) + t0
    obase = wid * ROWS_PER_W

    bufs = (buf0, buf1)
    sg = (sg0, sg1)
    ss = (ss0, ss1)

    gathers = [None] * NCH
    scatters = [None] * NCH
    for ci in range(NCH):
        p = ci % 2
        if ci >= 2:
            scatters[ci - 2].wait()
        gathers[ci] = pltpu.async_copy(
            weight_hbm.at[pl.ds(base + ci * CH, CH)], bufs[p], sg[p]
        )
        if ci >= 1:
            gathers[ci - 1].wait()
            scatters[ci - 1] = pltpu.async_copy(
                bufs[1 - p], out_hbm.at[pl.ds(obase + (ci - 1) * CH, CH)], ss[1 - p]
            )
    gathers[NCH - 1].wait()
    scatters[NCH - 1] = pltpu.async_copy(
        bufs[(NCH - 1) % 2],
        out_hbm.at[pl.ds(obase + (NCH - 1) * CH, CH)],
        ss[(NCH - 1) % 2],
    )
    scatters[NCH - 2].wait()
    scatters[NCH - 1].wait()


def kernel(length, offset, weight):
    del length
    off = jnp.ravel(offset).astype(jnp.int32)
    off_pad = jnp.zeros((L,), jnp.int32).at[:B].set(off)
    out = _sc_gather(off_pad, weight)
    return out.reshape(B, T, D)
